# baseline (speedup 1.0000x reference)
import functools

import jax
import jax.numpy as jnp
from jax import lax
from jax.experimental import pallas as pl
from jax.experimental.pallas import tpu as pltpu
from jax.experimental.pallas import tpu_sc as plsc

FEATDIM = 8192
OUTDIM = 8
WINDOWS_DAYS = [7, 30, 90, 365]
PRED_DAY_UNIX = 1577836800
THRESHOLDS = [PRED_DAY_UNIX - d * 86400 for d in WINDOWS_DAYS]
THR_ASC = THRESHOLDS[::-1]
NWIN = len(THRESHOLDS)
BSZ, SEQ, D2 = 256, 50, 16
EV = SEQ * D2
NGRP = EV // 16
NWORKERS = 32
PPW = BSZ // NWORKERS
INT_MIN = -2147483648
INT_MAX = 2147483647


def _sc_levels(conc, times_b):
    mesh = plsc.VectorSubcoreMesh(core_axis_name="c", subcore_axis_name="s")

    @functools.partial(
        pl.kernel,
        mesh=mesh,
        compiler_params=pltpu.CompilerParams(needs_layout_passes=False),
        out_type=(
            jax.ShapeDtypeStruct((BSZ * FEATDIM,), jnp.int32),
            jax.ShapeDtypeStruct((BSZ * 16,), jnp.int32),
        ),
        scratch_types=[
            pltpu.VMEM((PPW * EV,), jnp.int32),
            pltpu.VMEM((PPW * EV,), jnp.int32),
            pltpu.VMEM((2 * FEATDIM,), jnp.int32),
            pltpu.VMEM((PPW * 16,), jnp.int32),
            pltpu.SemaphoreType.DMA,
            pltpu.SemaphoreType.DMA,
        ],
    )
    def k(conc_hbm, times_hbm, lvl_hbm, tmm_hbm, conc_v, time_v, lvl_v, tmm_v,
          sem0, sem1):
        wid = lax.axis_index("s") * 2 + lax.axis_index("c")
        p0 = wid * PPW
        pltpu.sync_copy(conc_hbm.at[pl.ds(p0 * EV, PPW * EV)], conc_v)
        pltpu.sync_copy(times_hbm.at[pl.ds(p0 * EV, PPW * EV)], time_v)

        neg1 = jnp.full((16,), -1, jnp.int32)

        def init(j, carry):
            lvl_v[pl.ds(j * 16, 16)] = neg1
            return carry

        lax.fori_loop(0, 2 * FEATDIM // 16, init, 0)

        sems = (sem0, sem1)
        copies = [None, None]
        lane = lax.iota(jnp.int32, 16)
        for i in range(PPW):
            buf = i & 1
            if copies[buf] is not None:
                copies[buf].wait()
            p = p0 + i
            stamp = p * 8
            vals = [jnp.full((16,), stamp + kk, jnp.int32) for kk in (1, 2, 3, 4)]

            def grp0(g, carry):
                tmin, t0max = carry
                c = conc_v[pl.ds(i * EV + g * 16, 16)]
                t = time_v[pl.ds(i * EV + g * 16, 16)]
                coff = c + buf * FEATDIM
                plsc.store_scatter(lvl_v, [coff], vals[0], mask=t >= THR_ASC[0])
                tmin = jnp.minimum(tmin, t)
                t0max = jnp.maximum(t0max, jnp.where(c == 0, t, INT_MIN))
                return tmin, t0max

            tmin, t0max = lax.fori_loop(
                0, NGRP, grp0,
                (jnp.full((16,), INT_MAX, jnp.int32),
                 jnp.full((16,), INT_MIN, jnp.int32)),
            )

            for kk in range(1, NWIN):
                def grpk(g, carry, kk=kk):
                    c = conc_v[pl.ds(i * EV + g * 16, 16)]
                    t = time_v[pl.ds(i * EV + g * 16, 16)]
                    coff = c + buf * FEATDIM
                    plsc.store_scatter(
                        lvl_v, [coff], vals[kk], mask=t >= THR_ASC[kk]
                    )
                    return carry

                lax.fori_loop(0, NGRP, grpk, 0)
            tmn = jnp.min(tmin)
            t0m = jnp.max(t0max)
            tmm_v[pl.ds(i * 16, 16)] = jnp.where(
                lane == 0, tmn, jnp.where(lane == 1, t0m, 0)
            )
            copies[buf] = pltpu.async_copy(
                lvl_v.at[pl.ds(buf * FEATDIM, FEATDIM)],
                lvl_hbm.at[pl.ds(p * FEATDIM, FEATDIM)],
                sems[buf],
            )
        for cp in copies:
            if cp is not None:
                cp.wait()
        pltpu.sync_copy(tmm_v, tmm_hbm.at[pl.ds(p0 * 16, PPW * 16)])

    return k(conc, times_b)


def _tc_head(lvl, tmm, wc_list, w0, bias):
    kblk = 1024
    nk = FEATDIM // kblk

    def body(lvl_ref, w1, w2, w3, w4, tmm_ref, w0_ref, b_ref, o_ref, acc_ref):
        kk = pl.program_id(0)

        @pl.when(kk == 0)
        def _():
            acc_ref[...] = jnp.zeros_like(acc_ref)

        v = lvl_ref[...]
        prow = lax.broadcasted_iota(jnp.int32, (BSZ, kblk), 0)
        lev = jnp.where((v >> 3) == prow, v & 7, 0)
        acc = acc_ref[...]
        for ll, wref in ((1, w1), (2, w2), (3, w3), (4, w4)):
            m = (lev == ll).astype(jnp.bfloat16)
            acc += jnp.dot(m, wref[...], preferred_element_type=jnp.float32)
        acc_ref[...] = acc

        @pl.when(kk == nk - 1)
        def _():
            tmin = tmm_ref[:, 0:1]
            t0max = tmm_ref[:, 1:2]
            wcol = lax.broadcasted_iota(jnp.int32, (1, NWIN), 1)
            thrv = jnp.full((1, NWIN), THRESHOLDS[0], jnp.int32)
            for j in range(1, NWIN):
                thrv = jnp.where(wcol == j, THRESHOLDS[j], thrv)
            flags = ((tmin < thrv) | (t0max >= thrv)).astype(jnp.float32)
            o_ref[...] = (
                acc_ref[...]
                + jnp.dot(flags, w0_ref[...], preferred_element_type=jnp.float32)
                + b_ref[...]
            )

    wspec = pl.BlockSpec((kblk, OUTDIM), lambda k: (k, 0))
    return pl.pallas_call(
        body,
        grid=(nk,),
        in_specs=[
            pl.BlockSpec((BSZ, kblk), lambda k: (0, k)),
            wspec, wspec, wspec, wspec,
            pl.BlockSpec((BSZ, 16), lambda k: (0, 0)),
            pl.BlockSpec((NWIN, OUTDIM), lambda k: (0, 0)),
            pl.BlockSpec((1, OUTDIM), lambda k: (0, 0)),
        ],
        out_specs=pl.BlockSpec((BSZ, OUTDIM), lambda k: (0, 0)),
        out_shape=jax.ShapeDtypeStruct((BSZ, OUTDIM), jnp.float32),
        scratch_shapes=[pltpu.VMEM((BSZ, OUTDIM), jnp.float32)],
    )(lvl, *wc_list, tmm, w0, bias)


def kernel(concept_tensor, times, W, b):
    bsz = concept_tensor.shape[0]
    conc = concept_tensor.reshape(bsz * EV)
    times_b = jnp.broadcast_to(times[:, :, None], (bsz, SEQ, D2)).reshape(bsz * EV)

    wr = W.reshape(OUTDIM, NWIN, FEATDIM)
    cum = jnp.cumsum(wr[:, ::-1, :], axis=1)
    cum = cum.at[:, :, 0].set(0.0)
    wc_list = [cum[:, ll, :].T.astype(jnp.bfloat16) for ll in range(NWIN)]
    w0 = wr[:, :, 0].T

    lvl, tmm = _sc_levels(conc, times_b)
    lvl = lvl.reshape(BSZ, FEATDIM)
    tmm = tmm.reshape(BSZ, 16)
    return _tc_head(lvl, tmm, wc_list, w0, b.reshape(1, OUTDIM))


# speedup vs baseline: 1.2698x; 1.2698x over previous
import functools

import jax
import jax.numpy as jnp
from jax import lax
from jax.experimental import pallas as pl
from jax.experimental.pallas import tpu as pltpu
from jax.experimental.pallas import tpu_sc as plsc

FEATDIM = 8192
OUTDIM = 8
WINDOWS_DAYS = [7, 30, 90, 365]
PRED_DAY_UNIX = 1577836800
THRESHOLDS = [PRED_DAY_UNIX - d * 86400 for d in WINDOWS_DAYS]
THR_ASC = THRESHOLDS[::-1]
NWIN = len(THRESHOLDS)
BSZ, SEQ, D2 = 256, 50, 16
EV = SEQ * D2
NGRP = EV // 16
NWORKERS = 32
PPW = BSZ // NWORKERS
INT_MIN = -2147483648
INT_MAX = 2147483647


def _sc_levels(conc, times_b):
    mesh = plsc.VectorSubcoreMesh(core_axis_name="c", subcore_axis_name="s")

    @functools.partial(
        pl.kernel,
        mesh=mesh,
        compiler_params=pltpu.CompilerParams(needs_layout_passes=False),
        out_type=(
            jax.ShapeDtypeStruct((BSZ * FEATDIM,), jnp.int32),
            jax.ShapeDtypeStruct((BSZ * 16,), jnp.int32),
        ),
        scratch_types=[
            pltpu.VMEM((PPW * EV,), jnp.int32),
            pltpu.VMEM((PPW * EV,), jnp.int32),
            pltpu.VMEM((2 * FEATDIM,), jnp.int32),
            pltpu.VMEM((PPW * 16,), jnp.int32),
            pltpu.SemaphoreType.DMA,
            pltpu.SemaphoreType.DMA,
        ],
    )
    def k(conc_hbm, times_hbm, lvl_hbm, tmm_hbm, conc_v, time_v, lvl_v, tmm_v,
          sem0, sem1):
        wid = lax.axis_index("s") * 2 + lax.axis_index("c")
        p0 = wid * PPW
        pltpu.sync_copy(conc_hbm.at[pl.ds(p0 * EV, PPW * EV)], conc_v)
        pltpu.sync_copy(times_hbm.at[pl.ds(p0 * EV, PPW * EV)], time_v)

        neg1 = jnp.full((16,), -1, jnp.int32)

        def init(j, carry):
            lvl_v[pl.ds(j * 16, 16)] = neg1
            return carry

        lax.fori_loop(0, 2 * FEATDIM // 16, init, 0)

        sems = (sem0, sem1)
        copies = [None, None]
        lane = lax.iota(jnp.int32, 16)
        for i in range(PPW):
            buf = i & 1
            if copies[buf] is not None:
                copies[buf].wait()
            p = p0 + i
            stamp = p * 8
            vals = [jnp.full((16,), stamp + kk, jnp.int32) for kk in (1, 2, 3, 4)]

            def grp0(g, carry):
                tmin, t0max = carry
                c = conc_v[pl.ds(i * EV + g * 16, 16)]
                t = time_v[pl.ds(i * EV + g * 16, 16)]
                coff = c + buf * FEATDIM
                plsc.store_scatter(lvl_v, [coff], vals[0], mask=t >= THR_ASC[0])
                tmin = jnp.minimum(tmin, t)
                t0max = jnp.maximum(t0max, jnp.where(c == 0, t, INT_MIN))
                return tmin, t0max

            tmin, t0max = lax.fori_loop(
                0, NGRP, grp0,
                (jnp.full((16,), INT_MAX, jnp.int32),
                 jnp.full((16,), INT_MIN, jnp.int32)),
            )

            for kk in range(1, NWIN):
                def grpk(g, carry, kk=kk):
                    c = conc_v[pl.ds(i * EV + g * 16, 16)]
                    t = time_v[pl.ds(i * EV + g * 16, 16)]
                    coff = c + buf * FEATDIM
                    plsc.store_scatter(
                        lvl_v, [coff], vals[kk], mask=t >= THR_ASC[kk]
                    )
                    return carry

                lax.fori_loop(0, NGRP, grpk, 0)
            tmn = jnp.min(tmin)
            t0m = jnp.max(t0max)
            tmm_v[pl.ds(i * 16, 16)] = jnp.where(
                lane == 0, tmn, jnp.where(lane == 1, t0m, 0)
            )
            copies[buf] = pltpu.async_copy(
                lvl_v.at[pl.ds(buf * FEATDIM, FEATDIM)],
                lvl_hbm.at[pl.ds(p * FEATDIM, FEATDIM)],
                sems[buf],
            )
        for cp in copies:
            if cp is not None:
                cp.wait()
        pltpu.sync_copy(tmm_v, tmm_hbm.at[pl.ds(p0 * 16, PPW * 16)])

    return k(conc, times_b)


def _tc_head(lvl, tmm, w4d, w0, bias):
    kblk = 1024
    nk = FEATDIM // kblk
    dn = (((1,), (1,)), ((), ()))

    def body(lvl_ref, w1, w2, w3, w4, tmm_ref, w0_ref, b_ref, o_ref, acc_ref):
        kk = pl.program_id(0)

        @pl.when(kk == 0)
        def _():
            acc_ref[...] = jnp.zeros_like(acc_ref)

        v = lvl_ref[...]
        prow = lax.broadcasted_iota(jnp.int32, (BSZ, kblk), 0)
        col = lax.broadcasted_iota(jnp.int32, (BSZ, kblk), 1)
        lev = jnp.where((v >> 3) == prow, v & 7, 0)
        lev = jnp.where((col == 0) & (kk == 0), 0, lev)
        acc = acc_ref[...]
        cum = jnp.zeros((OUTDIM, kblk), jnp.float32)
        for ll, wref in ((1, w4), (2, w3), (3, w2), (4, w1)):
            cum = cum + wref[...]
            m = (lev == ll).astype(jnp.bfloat16)
            acc += lax.dot_general(
                m, cum.astype(jnp.bfloat16), dn,
                preferred_element_type=jnp.float32,
            )
        acc_ref[...] = acc

        @pl.when(kk == nk - 1)
        def _():
            tmin = tmm_ref[:, 0:1]
            t0max = tmm_ref[:, 1:2]
            wcol = lax.broadcasted_iota(jnp.int32, (1, NWIN), 1)
            thrv = jnp.full((1, NWIN), THRESHOLDS[0], jnp.int32)
            for j in range(1, NWIN):
                thrv = jnp.where(wcol == j, THRESHOLDS[j], thrv)
            flags = ((tmin < thrv) | (t0max >= thrv)).astype(jnp.float32)
            o_ref[...] = (
                acc_ref[...]
                + jnp.dot(flags, w0_ref[...], preferred_element_type=jnp.float32)
                + b_ref[...]
            )

    wspecs = [
        pl.BlockSpec((OUTDIM, kblk), lambda k, j=j: (0, j * nk + k))
        for j in range(NWIN)
    ]
    return pl.pallas_call(
        body,
        grid=(nk,),
        in_specs=[
            pl.BlockSpec((BSZ, kblk), lambda k: (0, k)),
            *wspecs,
            pl.BlockSpec((BSZ, 16), lambda k: (0, 0)),
            pl.BlockSpec((NWIN, OUTDIM), lambda k: (0, 0)),
            pl.BlockSpec((1, OUTDIM), lambda k: (0, 0)),
        ],
        out_specs=pl.BlockSpec((BSZ, OUTDIM), lambda k: (0, 0)),
        out_shape=jax.ShapeDtypeStruct((BSZ, OUTDIM), jnp.float32),
        scratch_shapes=[pltpu.VMEM((BSZ, OUTDIM), jnp.float32)],
    )(lvl, w4d, w4d, w4d, w4d, tmm, w0, bias)


def kernel(concept_tensor, times, W, b):
    bsz = concept_tensor.shape[0]
    conc = concept_tensor.reshape(bsz * EV)
    times_b = jnp.broadcast_to(times[:, :, None], (bsz, SEQ, D2)).reshape(bsz * EV)

    w0 = W.reshape(OUTDIM, NWIN, FEATDIM)[:, :, 0].T

    lvl, tmm = _sc_levels(conc, times_b)
    lvl = lvl.reshape(BSZ, FEATDIM)
    tmm = tmm.reshape(BSZ, 16)
    return _tc_head(lvl, tmm, W, w0, b.reshape(1, OUTDIM))
